# baseline (speedup 1.0000x reference)
import functools

import jax
import jax.numpy as jnp
from jax import lax
from jax.experimental import pallas as pl
from jax.experimental.pallas import tpu as pltpu
from jax.experimental.pallas import tpu_sc as plsc

N = 10000
E = 320000
HALF = 64
HD = 128
W = 128
NSUB = 16
NCORE = 2
WIN_PER_SUB = 160
WIN_TOTAL = WIN_PER_SUB * NSUB
E_PAD = WIN_TOTAL * W - E
IDX_BLK = 16
NPADROW = 64
RCHUNK = 632
ROW_BLOCK = 1000


def _relu(v):
    return jnp.maximum(v, 0.0)


def _dot(a, b):
    return jnp.dot(a, b, preferred_element_type=jnp.float32)


def _head_body(x_ref, w1_ref, b1_ref, w2_ref, b2_ref, out_ref):
    x = x_ref[...]
    a = _relu(_dot(x[:, :HALF], w1_ref[...]) + b1_ref[...])
    b = _relu(_dot(x[:, HALF:], w1_ref[...]) + b1_ref[...])
    h = _dot(a + b, w2_ref[...]) + b2_ref[...]
    out_ref[0] = h[:, :HD]
    out_ref[1] = h[:, HD:]


def _head(x, w1, b1, w2, b2):
    grid = (N // ROW_BLOCK,)
    return pl.pallas_call(
        _head_body,
        grid=grid,
        in_specs=[
            pl.BlockSpec((ROW_BLOCK, 2 * HALF), lambda i: (i, 0)),
            pl.BlockSpec(w1.shape, lambda i: (0, 0)),
            pl.BlockSpec(b1.shape, lambda i: (0, 0)),
            pl.BlockSpec(w2.shape, lambda i: (0, 0)),
            pl.BlockSpec(b2.shape, lambda i: (0, 0)),
        ],
        out_specs=pl.BlockSpec((2, ROW_BLOCK, HD), lambda i: (0, i, 0)),
        out_shape=jax.ShapeDtypeStruct((2, N, HD), jnp.float32),
        compiler_params=pltpu.CompilerParams(
            dimension_semantics=("parallel",)),
    )(x, w1, b1, w2, b2)


def _gin_body(h2_ref, wa_ref, ba_ref, wb_ref, bb_ref, out_ref):
    z = jnp.concatenate([h2_ref[0], h2_ref[1]], axis=1)
    y = _relu(_dot(z, wa_ref[...]) + ba_ref[...])
    o = _relu(_dot(y, wb_ref[...]) + bb_ref[...])
    out_ref[0] = o[:, :HD]
    out_ref[1] = o[:, HD:]


def _gin(h2, wa, ba, wb, bb):
    grid = (N // ROW_BLOCK,)
    return pl.pallas_call(
        _gin_body,
        grid=grid,
        in_specs=[
            pl.BlockSpec((2, ROW_BLOCK, HD), lambda i: (0, i, 0)),
            pl.BlockSpec(wa.shape, lambda i: (0, 0)),
            pl.BlockSpec(ba.shape, lambda i: (0, 0)),
            pl.BlockSpec(wb.shape, lambda i: (0, 0)),
            pl.BlockSpec(bb.shape, lambda i: (0, 0)),
        ],
        out_specs=pl.BlockSpec((2, ROW_BLOCK, HD), lambda i: (0, i, 0)),
        out_shape=jax.ShapeDtypeStruct((2, N, HD), jnp.float32),
        compiler_params=pltpu.CompilerParams(
            dimension_semantics=("parallel",)),
    )(h2, wa, ba, wb, bb)


def _tail_body(h2_ref, wa_ref, ba_ref, wb_ref, bb_ref,
               o1_ref, p1_ref, o2_ref, p2_ref, o3_ref, p3_ref, out_ref):
    z = jnp.concatenate([h2_ref[0], h2_ref[1]], axis=1)
    y = _relu(_dot(z, wa_ref[...]) + ba_ref[...])
    t = _relu(_dot(y, wb_ref[...]) + bb_ref[...])
    u = _relu(_dot(t, o1_ref[...]) + p1_ref[...])
    v = _relu(_dot(u, o2_ref[...]) + p2_ref[...])
    out_ref[...] = _dot(v, o3_ref[...]) + p3_ref[...]


def _tail(h2, wa, ba, wb, bb, o1, p1, o2, p2, o3, p3):
    grid = (N // ROW_BLOCK,)
    ws = [wa, ba, wb, bb, o1, p1, o2, p2, o3, p3]
    return pl.pallas_call(
        _tail_body,
        grid=grid,
        in_specs=[pl.BlockSpec((2, ROW_BLOCK, HD), lambda i: (0, i, 0))]
        + [pl.BlockSpec(w.shape, lambda i: (0,) * w.ndim) for w in ws],
        out_specs=pl.BlockSpec((ROW_BLOCK, 64), lambda i: (i, 0)),
        out_shape=jax.ShapeDtypeStruct((N, 64), jnp.float32),
        compiler_params=pltpu.CompilerParams(
            dimension_semantics=("parallel",)),
    )(h2, *ws)


def _sc_agg(h2flat, srcs2, dsts2):
    mesh = plsc.VectorSubcoreMesh(core_axis_name="c", subcore_axis_name="s")

    @functools.partial(
        pl.kernel,
        out_type=jax.ShapeDtypeStruct((NCORE * N, HD), jnp.float32),
        mesh=mesh,
        scratch_types=[
            pltpu.VMEM((2, IDX_BLK, W), jnp.int32),
            pltpu.VMEM((2, IDX_BLK, W), jnp.int32),
            pltpu.VMEM((W, HD), jnp.float32),
            pltpu.VMEM((W, HD), jnp.float32),
            pltpu.VMEM_SHARED((N + NPADROW, HD), jnp.float32),
            pltpu.SemaphoreType.DMA,
            pltpu.SemaphoreType.DMA,
            pltpu.SemaphoreType.DMA,
            pltpu.SemaphoreType.DMA,
            pltpu.SemaphoreType.DMA,
            pltpu.SemaphoreType.DMA,
        ],
    )
    def k(h_hbm, src_hbm, dst_hbm, out_hbm, src_v, dst_v, buf0, buf1, acc,
          sem0, sem1, ssem0, ssem1, isem_s, isem_d):
        c = lax.axis_index("c")
        s = lax.axis_index("s")
        r0 = jnp.where(s == NSUB - 1, N - RCHUNK, s * RCHUNK)
        pltpu.sync_copy(h_hbm.at[pl.ds(c * N + r0, RCHUNK)],
                        acc.at[pl.ds(r0, RCHUNK)])
        plsc.subcore_barrier()
        w0 = c * WIN_TOTAL + s * WIN_PER_SUB
        d0 = s * WIN_PER_SUB

        bufs = (buf0, buf1)
        sems = (sem0, sem1)
        ssems = (ssem0, ssem1)
        NBLK = WIN_PER_SUB // IDX_BLK

        def wait_scatter(b, p):
            pltpu.make_async_copy(bufs[b], acc.at[dst_v.at[p].at[0]],
                                  ssems[b]).wait()

        def fetch_idx(g, p):
            pltpu.async_copy(src_hbm.at[pl.ds(w0 + g * IDX_BLK, IDX_BLK)],
                             src_v.at[p], isem_s)
            pltpu.async_copy(dst_hbm.at[pl.ds(d0 + g * IDX_BLK, IDX_BLK)],
                             dst_v.at[p], isem_d)

        def wait_idx(g, p):
            pltpu.make_async_copy(src_hbm.at[pl.ds(w0 + g * IDX_BLK, IDX_BLK)],
                                  src_v.at[p], isem_s).wait()
            pltpu.make_async_copy(dst_hbm.at[pl.ds(d0 + g * IDX_BLK, IDX_BLK)],
                                  dst_v.at[p], isem_d).wait()

        fetch_idx(0, 0)

        def block(g, carry):
            p = lax.rem(g, 2)
            wait_idx(g, p)

            @pl.when(g > 0)
            def _():
                wait_scatter(0, p)
            sv = src_v.at[p]
            dv = dst_v.at[p]
            gd = [pltpu.async_copy(h_hbm.at[sv.at[0]], buf0, sem0), None]
            for j in range(IDX_BLK):
                cur = j % 2
                nxt = 1 - cur
                if j + 1 < IDX_BLK:
                    if j >= 1:
                        wait_scatter(nxt, p)
                    else:
                        @pl.when(g > 0)
                        def _():
                            wait_scatter(1, p)
                        @pl.when(g + 1 < NBLK)
                        def _():
                            fetch_idx(g + 1, 1 - p)
                    gd[nxt] = pltpu.async_copy(
                        h_hbm.at[sv.at[j + 1]], bufs[nxt], sems[nxt])
                gd[cur].wait()
                pltpu.async_copy(bufs[cur], acc.at[dv.at[j]], ssems[cur],
                                 add=True)
            return carry

        lax.fori_loop(0, NBLK, block, 0)
        wait_scatter(0, 0)
        wait_scatter(1, 0)
        plsc.subcore_barrier()
        pltpu.sync_copy(acc.at[pl.ds(r0, RCHUNK)],
                        out_hbm.at[pl.ds(c * N + r0, RCHUNK)])

    return k(h2flat, srcs2, dsts2)


def _fold_bn(lin, bn, scale):
    g = bn["gamma"] * scale
    return lin["W"] * g[None, :], lin["b"] * g + bn["beta"]


def kernel(x, edge_index, params):
    inv = 1.0 / jnp.sqrt(jnp.float32(1.0 + 1e-5))

    mlp = params["mlp"]
    w1, b1 = _fold_bn(mlp["lin1"], mlp["bn1"], inv)
    w2 = mlp["lin2"]["W"]
    b2 = 2.0 * mlp["lin2"]["b"]

    gin_w = []
    for lp in params["gin"]:
        wa, ba = _fold_bn(lp["lin1"], lp["bn"], inv)
        gin_w.append((wa, ba[None, :], lp["lin2"]["W"], lp["lin2"]["b"][None, :]))

    op = params["out"]
    o1, p1 = _fold_bn(op["lin1"], op["bn1"], inv)
    o2, p2 = _fold_bn(op["lin2"], op["bn2"], inv)
    o3, p3 = op["lin3"]["W"], op["lin3"]["b"]

    src = edge_index[0].astype(jnp.int32)
    dst = edge_index[1].astype(jnp.int32)
    pad = jnp.arange(E_PAD, dtype=jnp.int32)
    src = jnp.concatenate([src, pad % N])
    dst = jnp.concatenate([dst, pad % NPADROW + N])
    srcs2 = jnp.concatenate([src, src + N]).reshape(NCORE * WIN_TOTAL, W)
    dsts2 = dst.reshape(WIN_TOTAL, W)

    h2 = _head(x, w1, b1[None, :], w2, b2[None, :])
    for layer in range(3):
        acc = _sc_agg(h2.reshape(NCORE * N, HD), srcs2, dsts2)
        acc = acc.reshape(NCORE, N, HD)
        if layer < 2:
            h2 = _gin(acc, *gin_w[layer])
        else:
            wa, ba, wb, bb = gin_w[2]
            out = _tail(acc, wa, ba, wb, bb,
                        o1, p1[None, :], o2, p2[None, :], o3, p3[None, :])
    return out


# speedup vs baseline: 1.0204x; 1.0204x over previous
import functools

import jax
import jax.numpy as jnp
from jax import lax
from jax.experimental import pallas as pl
from jax.experimental.pallas import tpu as pltpu
from jax.experimental.pallas import tpu_sc as plsc

N = 10000
E = 320000
HALF = 64
HD = 128
W = 128
NSUB = 16
NCORE = 2
WIN_PER_SUB = 160
WIN_TOTAL = WIN_PER_SUB * NSUB
E_PAD = WIN_TOTAL * W - E
IDX_BLK = 16
NPADROW = 64
RCHUNK = 632
ROW_BLOCK = 2000


def _relu(v):
    return jnp.maximum(v, 0.0)


def _dot(a, b):
    return jnp.dot(a, b, preferred_element_type=jnp.float32)


def _head_body(x_ref, w1_ref, b1_ref, w2_ref, b2_ref, out_ref):
    x = x_ref[...]
    a = _relu(_dot(x[:, :HALF], w1_ref[...]) + b1_ref[...])
    b = _relu(_dot(x[:, HALF:], w1_ref[...]) + b1_ref[...])
    h = _dot(a + b, w2_ref[...]) + b2_ref[...]
    out_ref[0] = h[:, :HD]
    out_ref[1] = h[:, HD:]


def _head(x, w1, b1, w2, b2):
    grid = (N // ROW_BLOCK,)
    return pl.pallas_call(
        _head_body,
        grid=grid,
        in_specs=[
            pl.BlockSpec((ROW_BLOCK, 2 * HALF), lambda i: (i, 0)),
            pl.BlockSpec(w1.shape, lambda i: (0, 0)),
            pl.BlockSpec(b1.shape, lambda i: (0, 0)),
            pl.BlockSpec(w2.shape, lambda i: (0, 0)),
            pl.BlockSpec(b2.shape, lambda i: (0, 0)),
        ],
        out_specs=pl.BlockSpec((2, ROW_BLOCK, HD), lambda i: (0, i, 0)),
        out_shape=jax.ShapeDtypeStruct((2, N, HD), jnp.float32),
        compiler_params=pltpu.CompilerParams(
            dimension_semantics=("parallel",)),
    )(x, w1, b1, w2, b2)


def _gin_body(h2_ref, wa_ref, ba_ref, wb_ref, bb_ref, out_ref):
    z = jnp.concatenate([h2_ref[0], h2_ref[1]], axis=1)
    y = _relu(_dot(z, wa_ref[...]) + ba_ref[...])
    o = _relu(_dot(y, wb_ref[...]) + bb_ref[...])
    out_ref[0] = o[:, :HD]
    out_ref[1] = o[:, HD:]


def _gin(h2, wa, ba, wb, bb):
    grid = (N // ROW_BLOCK,)
    return pl.pallas_call(
        _gin_body,
        grid=grid,
        in_specs=[
            pl.BlockSpec((2, ROW_BLOCK, HD), lambda i: (0, i, 0)),
            pl.BlockSpec(wa.shape, lambda i: (0, 0)),
            pl.BlockSpec(ba.shape, lambda i: (0, 0)),
            pl.BlockSpec(wb.shape, lambda i: (0, 0)),
            pl.BlockSpec(bb.shape, lambda i: (0, 0)),
        ],
        out_specs=pl.BlockSpec((2, ROW_BLOCK, HD), lambda i: (0, i, 0)),
        out_shape=jax.ShapeDtypeStruct((2, N, HD), jnp.float32),
        compiler_params=pltpu.CompilerParams(
            dimension_semantics=("parallel",)),
    )(h2, wa, ba, wb, bb)


def _tail_body(h2_ref, wa_ref, ba_ref, wb_ref, bb_ref,
               o1_ref, p1_ref, o2_ref, p2_ref, o3_ref, p3_ref, out_ref):
    z = jnp.concatenate([h2_ref[0], h2_ref[1]], axis=1)
    y = _relu(_dot(z, wa_ref[...]) + ba_ref[...])
    t = _relu(_dot(y, wb_ref[...]) + bb_ref[...])
    u = _relu(_dot(t, o1_ref[...]) + p1_ref[...])
    v = _relu(_dot(u, o2_ref[...]) + p2_ref[...])
    out_ref[...] = _dot(v, o3_ref[...]) + p3_ref[...]


def _tail(h2, wa, ba, wb, bb, o1, p1, o2, p2, o3, p3):
    grid = (N // ROW_BLOCK,)
    ws = [wa, ba, wb, bb, o1, p1, o2, p2, o3, p3]
    return pl.pallas_call(
        _tail_body,
        grid=grid,
        in_specs=[pl.BlockSpec((2, ROW_BLOCK, HD), lambda i: (0, i, 0))]
        + [pl.BlockSpec(w.shape, lambda i: (0,) * w.ndim) for w in ws],
        out_specs=pl.BlockSpec((ROW_BLOCK, 64), lambda i: (i, 0)),
        out_shape=jax.ShapeDtypeStruct((N, 64), jnp.float32),
        compiler_params=pltpu.CompilerParams(
            dimension_semantics=("parallel",)),
    )(h2, *ws)


def _sc_agg(h2flat, srcs2, dsts2):
    mesh = plsc.VectorSubcoreMesh(core_axis_name="c", subcore_axis_name="s")

    @functools.partial(
        pl.kernel,
        out_type=jax.ShapeDtypeStruct((NCORE * N, HD), jnp.float32),
        mesh=mesh,
        scratch_types=[
            pltpu.VMEM((2, IDX_BLK, W), jnp.int32),
            pltpu.VMEM((2, IDX_BLK, W), jnp.int32),
            pltpu.VMEM((W, HD), jnp.float32),
            pltpu.VMEM((W, HD), jnp.float32),
            pltpu.VMEM_SHARED((N + NPADROW, HD), jnp.float32),
            pltpu.SemaphoreType.DMA,
            pltpu.SemaphoreType.DMA,
            pltpu.SemaphoreType.DMA,
            pltpu.SemaphoreType.DMA,
            pltpu.SemaphoreType.DMA,
            pltpu.SemaphoreType.DMA,
        ],
    )
    def k(h_hbm, src_hbm, dst_hbm, out_hbm, src_v, dst_v, buf0, buf1, acc,
          sem0, sem1, ssem0, ssem1, isem_s, isem_d):
        c = lax.axis_index("c")
        s = lax.axis_index("s")
        r0 = jnp.where(s == NSUB - 1, N - RCHUNK, s * RCHUNK)
        pltpu.sync_copy(h_hbm.at[pl.ds(c * N + r0, RCHUNK)],
                        acc.at[pl.ds(r0, RCHUNK)])
        plsc.subcore_barrier()
        w0 = c * WIN_TOTAL + s * WIN_PER_SUB
        d0 = s * WIN_PER_SUB

        bufs = (buf0, buf1)
        sems = (sem0, sem1)
        ssems = (ssem0, ssem1)
        NBLK = WIN_PER_SUB // IDX_BLK

        def wait_scatter(b, p):
            pltpu.make_async_copy(bufs[b], acc.at[dst_v.at[p].at[0]],
                                  ssems[b]).wait()

        def fetch_idx(g, p):
            pltpu.async_copy(src_hbm.at[pl.ds(w0 + g * IDX_BLK, IDX_BLK)],
                             src_v.at[p], isem_s)
            pltpu.async_copy(dst_hbm.at[pl.ds(d0 + g * IDX_BLK, IDX_BLK)],
                             dst_v.at[p], isem_d)

        def wait_idx(g, p):
            pltpu.make_async_copy(src_hbm.at[pl.ds(w0 + g * IDX_BLK, IDX_BLK)],
                                  src_v.at[p], isem_s).wait()
            pltpu.make_async_copy(dst_hbm.at[pl.ds(d0 + g * IDX_BLK, IDX_BLK)],
                                  dst_v.at[p], isem_d).wait()

        fetch_idx(0, 0)

        def block(g, carry):
            p = lax.rem(g, 2)
            wait_idx(g, p)

            @pl.when(g > 0)
            def _():
                wait_scatter(0, p)
            sv = src_v.at[p]
            dv = dst_v.at[p]
            gd = [pltpu.async_copy(h_hbm.at[sv.at[0]], buf0, sem0), None]
            for j in range(IDX_BLK):
                cur = j % 2
                nxt = 1 - cur
                if j + 1 < IDX_BLK:
                    if j >= 1:
                        wait_scatter(nxt, p)
                    else:
                        @pl.when(g > 0)
                        def _():
                            wait_scatter(1, p)
                        @pl.when(g + 1 < NBLK)
                        def _():
                            fetch_idx(g + 1, 1 - p)
                    gd[nxt] = pltpu.async_copy(
                        h_hbm.at[sv.at[j + 1]], bufs[nxt], sems[nxt])
                gd[cur].wait()
                pltpu.async_copy(bufs[cur], acc.at[dv.at[j]], ssems[cur],
                                 add=True)
            return carry

        lax.fori_loop(0, NBLK, block, 0)
        wait_scatter(0, 0)
        wait_scatter(1, 0)
        plsc.subcore_barrier()
        pltpu.sync_copy(acc.at[pl.ds(r0, RCHUNK)],
                        out_hbm.at[pl.ds(c * N + r0, RCHUNK)])

    return k(h2flat, srcs2, dsts2)


def _fold_bn(lin, bn, scale):
    g = bn["gamma"] * scale
    return lin["W"] * g[None, :], lin["b"] * g + bn["beta"]


def kernel(x, edge_index, params):
    inv = 1.0 / jnp.sqrt(jnp.float32(1.0 + 1e-5))

    mlp = params["mlp"]
    w1, b1 = _fold_bn(mlp["lin1"], mlp["bn1"], inv)
    w2 = mlp["lin2"]["W"]
    b2 = 2.0 * mlp["lin2"]["b"]

    gin_w = []
    for lp in params["gin"]:
        wa, ba = _fold_bn(lp["lin1"], lp["bn"], inv)
        gin_w.append((wa, ba[None, :], lp["lin2"]["W"], lp["lin2"]["b"][None, :]))

    op = params["out"]
    o1, p1 = _fold_bn(op["lin1"], op["bn1"], inv)
    o2, p2 = _fold_bn(op["lin2"], op["bn2"], inv)
    o3, p3 = op["lin3"]["W"], op["lin3"]["b"]

    src = edge_index[0].astype(jnp.int32)
    dst = edge_index[1].astype(jnp.int32)
    pad = jnp.arange(E_PAD, dtype=jnp.int32)
    src = jnp.concatenate([src, pad % N])
    dst = jnp.concatenate([dst, pad % NPADROW + N])
    srcs2 = jnp.concatenate([src, src + N]).reshape(NCORE * WIN_TOTAL, W)
    dsts2 = dst.reshape(WIN_TOTAL, W)

    h2 = _head(x, w1, b1[None, :], w2, b2[None, :])
    for layer in range(3):
        acc = _sc_agg(h2.reshape(NCORE * N, HD), srcs2, dsts2)
        acc = acc.reshape(NCORE, N, HD)
        if layer < 2:
            h2 = _gin(acc, *gin_w[layer])
        else:
            wa, ba, wb, bb = gin_w[2]
            out = _tail(acc, wa, ba, wb, bb,
                        o1, p1[None, :], o2, p2[None, :], o3, p3[None, :])
    return out
